# baseline (speedup 1.0000x reference)
import jax
import jax.numpy as jnp
from jax import lax
from jax.experimental import pallas as pl
from jax.experimental.pallas import tpu as pltpu

_N = 4194304
_GRID = 32
_BLOCK = _N // _GRID


def _body(c_ref, w_ref, o_ref):
    o_ref[...] = w_ref[...]

    @pl.when(pl.program_id(0) == 0)
    def _patch():
        a = c_ref[9] / (c_ref[10] * 400000.0) * 0.001
        b = c_ref[11] / c_ref[10]
        head = w_ref[pl.ds(0, 128)]
        idx = lax.broadcasted_iota(jnp.int32, head.shape, 0)
        head = jnp.where(idx == 0, a, head)
        head = jnp.where(idx == 1, b, head)
        o_ref[pl.ds(0, 128)] = head


def kernel(y, w, c, t):
    return pl.pallas_call(
        _body,
        grid=(_GRID,),
        in_specs=[
            pl.BlockSpec(memory_space=pltpu.SMEM),
            pl.BlockSpec((_BLOCK,), lambda i: (i,)),
        ],
        out_specs=pl.BlockSpec((_BLOCK,), lambda i: (i,)),
        out_shape=jax.ShapeDtypeStruct((_N,), jnp.float32),
    )(c, w)


# speedup vs baseline: 1.8159x; 1.8159x over previous
import jax
import jax.numpy as jnp
from jax import lax
from jax.experimental import pallas as pl
from jax.experimental.pallas import tpu as pltpu

_N = 4194304
_GRID = 8
_BLOCK = _N // _GRID


def _body(c_ref, w_ref, o_ref):
    o_ref[...] = w_ref[...]

    @pl.when(pl.program_id(0) == 0)
    def _patch():
        a = c_ref[9] / (c_ref[10] * 400000.0) * 0.001
        b = c_ref[11] / c_ref[10]
        head = w_ref[pl.ds(0, 128)]
        idx = lax.broadcasted_iota(jnp.int32, head.shape, 0)
        head = jnp.where(idx == 0, a, head)
        head = jnp.where(idx == 1, b, head)
        o_ref[pl.ds(0, 128)] = head


def kernel(y, w, c, t):
    return pl.pallas_call(
        _body,
        grid=(_GRID,),
        in_specs=[
            pl.BlockSpec(memory_space=pltpu.SMEM),
            pl.BlockSpec((_BLOCK,), lambda i: (i,)),
        ],
        out_specs=pl.BlockSpec((_BLOCK,), lambda i: (i,)),
        out_shape=jax.ShapeDtypeStruct((_N,), jnp.float32),
    )(c, w)


# speedup vs baseline: 1.9711x; 1.0854x over previous
import jax
import jax.numpy as jnp
from jax import lax
from jax.experimental import pallas as pl
from jax.experimental.pallas import tpu as pltpu

_N = 4194304
_GRID = 4
_BLOCK = _N // _GRID


def _body(c_ref, w_ref, o_ref):
    o_ref[...] = w_ref[...]

    @pl.when(pl.program_id(0) == 0)
    def _patch():
        a = c_ref[9] / (c_ref[10] * 400000.0) * 0.001
        b = c_ref[11] / c_ref[10]
        head = w_ref[pl.ds(0, 128)]
        idx = lax.broadcasted_iota(jnp.int32, head.shape, 0)
        head = jnp.where(idx == 0, a, head)
        head = jnp.where(idx == 1, b, head)
        o_ref[pl.ds(0, 128)] = head


def kernel(y, w, c, t):
    return pl.pallas_call(
        _body,
        grid=(_GRID,),
        in_specs=[
            pl.BlockSpec(memory_space=pltpu.SMEM),
            pl.BlockSpec((_BLOCK,), lambda i: (i,)),
        ],
        out_specs=pl.BlockSpec((_BLOCK,), lambda i: (i,)),
        out_shape=jax.ShapeDtypeStruct((_N,), jnp.float32),
    )(c, w)


# speedup vs baseline: 2.2506x; 1.1418x over previous
import jax
import jax.numpy as jnp
from jax import lax
from jax.experimental import pallas as pl
from jax.experimental.pallas import tpu as pltpu

_N = 4194304
_GRID = 2
_BLOCK = _N // _GRID


def _body(c_ref, w_ref, o_ref):
    o_ref[...] = w_ref[...]

    @pl.when(pl.program_id(0) == 0)
    def _patch():
        a = c_ref[9] / (c_ref[10] * 400000.0) * 0.001
        b = c_ref[11] / c_ref[10]
        head = w_ref[pl.ds(0, 128)]
        idx = lax.broadcasted_iota(jnp.int32, head.shape, 0)
        head = jnp.where(idx == 0, a, head)
        head = jnp.where(idx == 1, b, head)
        o_ref[pl.ds(0, 128)] = head


def kernel(y, w, c, t):
    return pl.pallas_call(
        _body,
        grid=(_GRID,),
        in_specs=[
            pl.BlockSpec(memory_space=pltpu.SMEM),
            pl.BlockSpec((_BLOCK,), lambda i: (i,)),
        ],
        out_specs=pl.BlockSpec((_BLOCK,), lambda i: (i,)),
        out_shape=jax.ShapeDtypeStruct((_N,), jnp.float32),
    )(c, w)
